# Initial kernel scaffold; baseline (speedup 1.0000x reference)
import jax
import jax.numpy as jnp
from jax.experimental import pallas as pl


def kernel(x_author, x_paper, ptr_author, ptr_paper, W_author, b_author, W_paper, b_paper):
    raise NotImplementedError("write your pallas kernel here")


# speedup vs baseline: 31.7965x; 31.7965x over previous
import jax
import jax.numpy as jnp
from jax import lax
from jax.experimental import pallas as pl
from jax.experimental.pallas import tpu as pltpu
from jax.experimental.pallas import tpu_sc as plsc

_LANES = 16


def _build_tables(ptr, n, s, n_workers, seg_per_w, tw, tw2):
    nseg_pad = n_workers * seg_per_w
    ptr_pad = jnp.full((nseg_pad + 1,), n, jnp.int32).at[:s + 1].set(
        ptr.astype(jnp.int32))
    seg_lo = ptr_pad[:-1]
    seg_hi = ptr_pad[1:]
    nonempty = seg_hi > seg_lo
    segs = jnp.arange(nseg_pad, dtype=jnp.int32)
    w_of = segs // seg_per_w
    ex = jnp.cumsum(nonempty.astype(jnp.int32)) - nonempty.astype(jnp.int32)
    local_pos = ex - ex[w_of * seg_per_w]
    dump = n_workers * tw2
    tgt = jnp.where(nonempty, w_of * tw2 + local_pos, dump)
    his = jnp.zeros((dump + 1,), jnp.int32).at[tgt].set(seg_hi)
    ids = jnp.zeros((dump + 1,), jnp.int32).at[tgt].set(
        segs - w_of * seg_per_w)
    wrows = jnp.arange(n_workers, dtype=jnp.int32)
    his = his.at[wrows * tw2 + tw].set(ptr_pad[wrows * seg_per_w])
    his = his.at[wrows * tw2 + tw + 1].set(ptr_pad[wrows * seg_per_w
                                                   + seg_per_w])
    return his[:dump], ids[:dump]


def _sc_segmax(x_a, x_p, tab_a, tab_p, *, n, d, seg_per_w, chunk,
               n_workers, tw, tw2):
    nseg_pad = n_workers * seg_per_w
    kg = d // _LANES
    mesh = plsc.VectorSubcoreMesh(core_axis_name="c", subcore_axis_name="s")
    num_cores = 2

    def pscal(ref, i):
        return ref[pl.ds(i, _LANES)][0]

    def body(xa_hbm, xp_hbm, ha_hbm, ia_hbm, hp_hbm, ip_hbm, ma_hbm, mp_hbm,
             hi_v, id_v, buf_v, m_v):
        wid = lax.axis_index("s") * num_cores + lax.axis_index("c")
        s0 = pl.multiple_of(wid * seg_per_w, 8)
        t0 = pl.multiple_of(wid * tw2, 8)
        neg = jnp.full((_LANES,), -jnp.inf, jnp.float32)
        zero = jnp.zeros((_LANES,), jnp.float32)

        for x_hbm, h_hbm, i_hbm, m_hbm in (
                (xa_hbm, ha_hbm, ia_hbm, ma_hbm),
                (xp_hbm, hp_hbm, ip_hbm, mp_hbm)):
            pltpu.sync_copy(h_hbm.at[pl.ds(t0, tw2)], hi_v)
            pltpu.sync_copy(i_hbm.at[pl.ds(t0, tw2)], id_v)
            rlo = pscal(hi_v, tw)
            rhi = pscal(hi_v, tw + 1)
            nch = jnp.maximum((rhi - rlo + chunk - 1) // chunk, 1)

            def zbody(j, _):
                for q in range(kg):
                    m_v[j, pl.ds(_LANES * q, _LANES)] = zero
                return 0
            lax.fori_loop(0, seg_per_w, zbody, 0)

            def chunk_body(c, carry):
                clo = rlo + c * chunk
                base = pl.multiple_of(
                    jnp.minimum(clo - clo % 8, n - (chunk + 8)), 8)
                clim = jnp.minimum(clo + chunk, rhi)
                pltpu.sync_copy(x_hbm.at[pl.ds(base, chunk + 8), :], buf_v)

                def row_body(g, st):
                    k, hi_cur = st[0], st[1]
                    acc = tuple(st[2:])
                    idx = g - base
                    acc = tuple(
                        jnp.maximum(acc[q],
                                    buf_v[idx, pl.ds(_LANES * q, _LANES)])
                        for q in range(kg))

                    def fin(*ops):
                        k_, _ = ops[0], ops[1]
                        a = ops[2:]
                        s_local = pscal(id_v, k_)
                        for q in range(kg):
                            m_v[s_local, pl.ds(_LANES * q, _LANES)] = a[q]
                        k2 = k_ + 1
                        return (k2, pscal(hi_v, k2)) + tuple(
                            neg for _ in range(kg))

                    def noop(*ops):
                        return tuple(ops)

                    return lax.cond(g + 1 == hi_cur, fin, noop,
                                    k, hi_cur, *acc)

                return lax.fori_loop(clo, clim, row_body, carry)

            init = (jnp.int32(0), pscal(hi_v, 0)) + tuple(
                neg for _ in range(kg))
            lax.fori_loop(0, nch, chunk_body, init)
            pltpu.sync_copy(m_v, m_hbm.at[pl.ds(s0, seg_per_w), :])

    f = pl.kernel(
        body,
        out_type=(
            jax.ShapeDtypeStruct((nseg_pad, d), jnp.float32),
            jax.ShapeDtypeStruct((nseg_pad, d), jnp.float32),
        ),
        mesh=mesh,
        scratch_types=[
            pltpu.VMEM((tw2,), jnp.int32),
            pltpu.VMEM((tw2,), jnp.int32),
            pltpu.VMEM((chunk + 8, d), jnp.float32),
            pltpu.VMEM((seg_per_w, d), jnp.float32),
        ],
    )
    return f(x_a, x_p, tab_a[0], tab_a[1], tab_p[0], tab_p[1])


def _tc_proj_reduce(m_a, m_p, W_a, b_a, W_p, b_p, *, nseg, d, out_dim, blk):
    ngrid = nseg // blk

    def body(ma, mp, wa, ba, wp, bp, out):
        i = pl.program_id(0)
        ya = jnp.dot(ma[...], wa[...], preferred_element_type=jnp.float32)
        yp = jnp.dot(mp[...], wp[...], preferred_element_type=jnp.float32)
        cand = jnp.maximum(
            jnp.max(ya, axis=0, keepdims=True) + ba[...],
            jnp.max(yp, axis=0, keepdims=True) + bp[...],
        )
        cand = jnp.maximum(cand, 0.0)

        @pl.when(i == 0)
        def _():
            out[...] = cand

        @pl.when(i > 0)
        def _():
            out[...] = jnp.maximum(out[...], cand)

    return pl.pallas_call(
        body,
        grid=(ngrid,),
        in_specs=[
            pl.BlockSpec((blk, d), lambda i: (i, 0)),
            pl.BlockSpec((blk, d), lambda i: (i, 0)),
            pl.BlockSpec((d, out_dim), lambda i: (0, 0)),
            pl.BlockSpec((1, out_dim), lambda i: (0, 0)),
            pl.BlockSpec((d, out_dim), lambda i: (0, 0)),
            pl.BlockSpec((1, out_dim), lambda i: (0, 0)),
        ],
        out_specs=pl.BlockSpec((1, out_dim), lambda i: (0, 0)),
        out_shape=jax.ShapeDtypeStruct((1, out_dim), jnp.float32),
    )(m_a, m_p, W_a, b_a.reshape(1, -1), W_p, b_p.reshape(1, -1))


def kernel(x_author, x_paper, ptr_author, ptr_paper,
           W_author, b_author, W_paper, b_paper):
    n, d = x_author.shape
    s = ptr_author.shape[0] - 1
    out_dim = W_author.shape[1]
    n_workers = 32
    seg_per_w = (-(-s // n_workers) + 7) // 8 * 8
    tw = seg_per_w + 8
    tw2 = tw + 24
    chunk = 256

    tab_a = _build_tables(ptr_author, n, s, n_workers, seg_per_w, tw, tw2)
    tab_p = _build_tables(ptr_paper, n, s, n_workers, seg_per_w, tw, tw2)

    m_a, m_p = _sc_segmax(x_author, x_paper, tab_a, tab_p,
                          n=n, d=d, seg_per_w=seg_per_w, chunk=chunk,
                          n_workers=n_workers, tw=tw, tw2=tw2)

    blk = 1000 if s % 1000 == 0 else 8
    out = _tc_proj_reduce(m_a, m_p, W_author, b_author, W_paper, b_paper,
                          nseg=s, d=d, out_dim=out_dim, blk=blk)
    return out.reshape(out_dim)


# speedup vs baseline: 99.6761x; 3.1348x over previous
import jax
import jax.numpy as jnp
from jax import lax
from jax.experimental import pallas as pl
from jax.experimental.pallas import tpu as pltpu
from jax.experimental.pallas import tpu_sc as plsc

_LANES = 16


def _sc_segmax(x_a, x_p, ptr_a, ptr_p, *, n, d, seg_per_w, chunk,
               n_workers, tw):
    nseg_pad = n_workers * seg_per_w
    kg = d // _LANES
    mesh = plsc.VectorSubcoreMesh(core_axis_name="c", subcore_axis_name="s")
    num_cores = 2
    bsteps = max(1, (tw - 1).bit_length())

    def pscal(ref, i):
        return ref[pl.ds(i, _LANES)][0]

    def body(xa_hbm, xp_hbm, pa_hbm, pp_hbm, ma_hbm, mp_hbm,
             ptr_v, buf_v, m_v, his_s, ids_s):
        wid = lax.axis_index("s") * num_cores + lax.axis_index("c")
        s0 = pl.multiple_of(wid * seg_per_w, 8)
        neg = jnp.full((_LANES,), -jnp.inf, jnp.float32)
        zero = jnp.zeros((_LANES,), jnp.float32)

        def reduce_rows(base, g0, g1, acc):
            def rbody(g, a):
                idx = g - base
                return tuple(
                    jnp.maximum(a[q], buf_v[idx, pl.ds(_LANES * q, _LANES)])
                    for q in range(kg))
            return lax.fori_loop(g0, g1, rbody, acc)

        for x_hbm, p_hbm, m_hbm in ((xa_hbm, pa_hbm, ma_hbm),
                                    (xp_hbm, pp_hbm, mp_hbm)):
            pltpu.sync_copy(p_hbm.at[pl.ds(s0, seg_per_w + 24)], ptr_v)
            rlo = pscal(ptr_v, 0)
            rhi = pscal(ptr_v, seg_per_w)

            def tinit(j, _):
                his_s[j] = n + 1
                return 0
            lax.fori_loop(0, tw, tinit, 0)

            def tbody(s_, carry):
                k_, lo_ = carry
                hi_ = pscal(ptr_v, s_ + 1)

                def t_yes(k__):
                    his_s[k__] = hi_
                    ids_s[k__] = s_
                    return k__ + 1

                k_ = lax.cond(hi_ > lo_, t_yes, lambda k__: k__, k_)
                return (k_, hi_)

            lax.fori_loop(0, seg_per_w, tbody, (jnp.int32(0), rlo))

            def zbody(j, _):
                for q in range(kg):
                    m_v[j, pl.ds(_LANES * q, _LANES)] = zero
                return 0
            lax.fori_loop(0, seg_per_w, zbody, 0)

            nch = jnp.maximum((rhi - rlo + chunk - 1) // chunk, 1)

            def chunk_body(c, carry):
                klo, r = carry[0], carry[1]
                acc = tuple(carry[2:])
                clo = rlo + c * chunk
                base = pl.multiple_of(
                    jnp.minimum(clo - clo % 8, n - (chunk + 8)), 8)
                clim = jnp.minimum(clo + chunk, rhi)
                pltpu.sync_copy(x_hbm.at[pl.ds(base, chunk + 8), :], buf_v)

                blo = jnp.int32(0)
                bhi = jnp.int32(tw)
                for _ in range(bsteps):
                    mid = (blo + bhi) // 2
                    gt = his_s[mid] > clim
                    live = blo < bhi
                    blo = jnp.where(jnp.logical_and(live, ~gt), mid + 1, blo)
                    bhi = jnp.where(jnp.logical_and(live, gt), mid, bhi)
                khi = blo

                def seg_body(k, st):
                    start = st[0]
                    a = tuple(st[1:])
                    hi_k = his_s[k]
                    a = reduce_rows(base, start, hi_k, a)
                    row = ids_s[k]
                    for q in range(kg):
                        m_v[row, pl.ds(_LANES * q, _LANES)] = a[q]
                    return (hi_k,) + tuple(neg for _ in range(kg))

                st = lax.fori_loop(klo, khi, seg_body, (r,) + acc)
                start = st[0]
                acc = tuple(st[1:])
                acc = reduce_rows(base, start, clim, acc)
                return (khi, clim) + acc

            init = (jnp.int32(0), rlo) + tuple(neg for _ in range(kg))
            lax.fori_loop(0, nch, chunk_body, init)
            pltpu.sync_copy(m_v, m_hbm.at[pl.ds(s0, seg_per_w), :])

    f = pl.kernel(
        body,
        out_type=(
            jax.ShapeDtypeStruct((nseg_pad, d), jnp.float32),
            jax.ShapeDtypeStruct((nseg_pad, d), jnp.float32),
        ),
        mesh=mesh,
        scratch_types=[
            pltpu.VMEM((seg_per_w + 24,), jnp.int32),
            pltpu.VMEM((chunk + 8, d), jnp.float32),
            pltpu.VMEM((seg_per_w, d), jnp.float32),
            pltpu.SMEM((tw,), jnp.int32),
            pltpu.SMEM((tw,), jnp.int32),
        ],
    )
    return f(x_a, x_p, ptr_a, ptr_p)


def _tc_proj_reduce(m_a, m_p, W_a, b_a, W_p, b_p, *, nseg, d, out_dim, blk):
    ngrid = nseg // blk

    def body(ma, mp, wa, ba, wp, bp, out):
        i = pl.program_id(0)
        ya = jnp.dot(ma[...], wa[...], preferred_element_type=jnp.float32)
        yp = jnp.dot(mp[...], wp[...], preferred_element_type=jnp.float32)
        cand = jnp.maximum(
            jnp.max(ya, axis=0, keepdims=True) + ba[...],
            jnp.max(yp, axis=0, keepdims=True) + bp[...],
        )
        cand = jnp.maximum(cand, 0.0)

        @pl.when(i == 0)
        def _():
            out[...] = cand

        @pl.when(i > 0)
        def _():
            out[...] = jnp.maximum(out[...], cand)

    return pl.pallas_call(
        body,
        grid=(ngrid,),
        in_specs=[
            pl.BlockSpec((blk, d), lambda i: (i, 0)),
            pl.BlockSpec((blk, d), lambda i: (i, 0)),
            pl.BlockSpec((d, out_dim), lambda i: (0, 0)),
            pl.BlockSpec((1, out_dim), lambda i: (0, 0)),
            pl.BlockSpec((d, out_dim), lambda i: (0, 0)),
            pl.BlockSpec((1, out_dim), lambda i: (0, 0)),
        ],
        out_specs=pl.BlockSpec((1, out_dim), lambda i: (0, 0)),
        out_shape=jax.ShapeDtypeStruct((1, out_dim), jnp.float32),
    )(m_a, m_p, W_a, b_a.reshape(1, -1), W_p, b_p.reshape(1, -1))


def kernel(x_author, x_paper, ptr_author, ptr_paper,
           W_author, b_author, W_paper, b_paper):
    n, d = x_author.shape
    s = ptr_author.shape[0] - 1
    out_dim = W_author.shape[1]
    n_workers = 32
    seg_per_w = (-(-s // n_workers) + 7) // 8 * 8
    nseg_pad = n_workers * seg_per_w
    tw = seg_per_w + 24
    chunk = 256

    pad = jnp.full((nseg_pad + 24 - (s + 1),), n, jnp.int32)
    ptr_a = jnp.concatenate([ptr_author.astype(jnp.int32), pad])
    ptr_p = jnp.concatenate([ptr_paper.astype(jnp.int32), pad])

    m_a, m_p = _sc_segmax(x_author, x_paper, ptr_a, ptr_p,
                          n=n, d=d, seg_per_w=seg_per_w, chunk=chunk,
                          n_workers=n_workers, tw=tw)

    blk = 1000 if s % 1000 == 0 else 8
    out = _tc_proj_reduce(m_a, m_p, W_author, b_author, W_paper, b_paper,
                          nseg=s, d=d, out_dim=out_dim, blk=blk)
    return out.reshape(out_dim)


# speedup vs baseline: 137.6451x; 1.3809x over previous
import jax
import jax.numpy as jnp
from jax import lax
from jax.experimental import pallas as pl
from jax.experimental.pallas import tpu as pltpu
from jax.experimental.pallas import tpu_sc as plsc

_LANES = 16


def _sc_segmax(x_a, x_p, ptr_a, ptr_p, *, n, d, seg_per_w, chunk,
               n_workers, tw):
    nseg_pad = n_workers * seg_per_w
    kg = d // _LANES
    mesh = plsc.VectorSubcoreMesh(core_axis_name="c", subcore_axis_name="s")
    num_cores = 2
    bsteps = max(1, (tw - 1).bit_length())

    def pscal(ref, i):
        return ref[pl.ds(i, _LANES)][0]

    def body(xa_hbm, xp_hbm, pa_hbm, pp_hbm, ma_hbm, mp_hbm,
             ptr_v, buf0_v, buf1_v, m_v, his_s, ids_s, sem0, sem1):
        wid = lax.axis_index("s") * num_cores + lax.axis_index("c")
        s0 = pl.multiple_of(wid * seg_per_w, 8)
        neg = jnp.full((_LANES,), -jnp.inf, jnp.float32)
        zero = jnp.zeros((_LANES,), jnp.float32)

        def reduce_rows(buf_v, base, g0, g1, acc):
            def rbody(g, a):
                idx = g - base
                return tuple(
                    jnp.maximum(a[q], buf_v[idx, pl.ds(_LANES * q, _LANES)])
                    for q in range(kg))
            return lax.fori_loop(g0, g1, rbody, acc)

        for x_hbm, p_hbm, m_hbm in ((xa_hbm, pa_hbm, ma_hbm),
                                    (xp_hbm, pp_hbm, mp_hbm)):
            pltpu.sync_copy(p_hbm.at[pl.ds(s0, seg_per_w + 24)], ptr_v)
            rlo = pscal(ptr_v, 0)
            rhi = pscal(ptr_v, seg_per_w)

            def tinit(j, _):
                his_s[j] = n + 1
                return 0
            lax.fori_loop(0, tw, tinit, 0)

            def tbody(s_, carry):
                k_, lo_ = carry
                hi_ = pscal(ptr_v, s_ + 1)

                def t_yes(k__):
                    his_s[k__] = hi_
                    ids_s[k__] = s_
                    return k__ + 1

                k_ = lax.cond(hi_ > lo_, t_yes, lambda k__: k__, k_)
                return (k_, hi_)

            lax.fori_loop(0, seg_per_w, tbody, (jnp.int32(0), rlo))

            def zbody(j, _):
                for q in range(kg):
                    m_v[j, pl.ds(_LANES * q, _LANES)] = zero
                return 0
            lax.fori_loop(0, seg_per_w, zbody, 0)

            nch = jnp.maximum((rhi - rlo + chunk - 1) // chunk, 1)

            def chunk_base(c):
                clo = rlo + c * chunk
                return pl.multiple_of(
                    jnp.minimum(clo - clo % 8, n - (chunk + 8)), 8)

            def dma_start(c, buf, sem):
                pltpu.make_async_copy(
                    x_hbm.at[pl.ds(chunk_base(c), chunk + 8), :], buf,
                    sem).start()

            def dma_wait(buf, sem):
                pltpu.make_async_copy(
                    x_hbm.at[pl.ds(0, chunk + 8), :], buf, sem).wait()

            def process(c, buf_v, carry):
                klo, r = carry[0], carry[1]
                acc = tuple(carry[2:])
                base = chunk_base(c)
                clim = jnp.minimum(rlo + c * chunk + chunk, rhi)

                blo = jnp.int32(0)
                bhi = jnp.int32(tw)
                for _ in range(bsteps):
                    mid = (blo + bhi) // 2
                    gt = his_s[mid] > clim
                    live = blo < bhi
                    blo = jnp.where(jnp.logical_and(live, ~gt), mid + 1, blo)
                    bhi = jnp.where(jnp.logical_and(live, gt), mid, bhi)
                khi = blo

                def seg_body(k, st):
                    start = st[0]
                    a = tuple(st[1:])
                    hi_k = his_s[k]
                    a = reduce_rows(buf_v, base, start, hi_k, a)
                    row = ids_s[k]
                    for q in range(kg):
                        m_v[row, pl.ds(_LANES * q, _LANES)] = a[q]
                    return (hi_k,) + tuple(neg for _ in range(kg))

                st = lax.fori_loop(klo, khi, seg_body, (r,) + acc)
                start = st[0]
                acc = tuple(st[1:])
                acc = reduce_rows(buf_v, base, start, clim, acc)
                return (khi, jnp.maximum(clim, r)) + acc

            nh = (nch + 1) // 2
            dma_start(0, buf0_v, sem0)

            def pair_body(h, carry):
                dma_start(2 * h + 1, buf1_v, sem1)
                dma_wait(buf0_v, sem0)
                carry = process(2 * h, buf0_v, carry)
                dma_start(2 * h + 2, buf0_v, sem0)
                dma_wait(buf1_v, sem1)
                carry = process(2 * h + 1, buf1_v, carry)
                return carry

            init = (jnp.int32(0), rlo) + tuple(neg for _ in range(kg))
            lax.fori_loop(0, nh, pair_body, init)
            dma_wait(buf0_v, sem0)
            pltpu.sync_copy(m_v, m_hbm.at[pl.ds(s0, seg_per_w), :])

    f = pl.kernel(
        body,
        out_type=(
            jax.ShapeDtypeStruct((nseg_pad, d), jnp.float32),
            jax.ShapeDtypeStruct((nseg_pad, d), jnp.float32),
        ),
        mesh=mesh,
        scratch_types=[
            pltpu.VMEM((seg_per_w + 24,), jnp.int32),
            pltpu.VMEM((chunk + 8, d), jnp.float32),
            pltpu.VMEM((chunk + 8, d), jnp.float32),
            pltpu.VMEM((seg_per_w, d), jnp.float32),
            pltpu.SMEM((tw,), jnp.int32),
            pltpu.SMEM((tw,), jnp.int32),
            pltpu.SemaphoreType.DMA,
            pltpu.SemaphoreType.DMA,
        ],
    )
    return f(x_a, x_p, ptr_a, ptr_p)


def _tc_proj_reduce(m_a, m_p, W_a, b_a, W_p, b_p, *, nseg, d, out_dim, blk):
    ngrid = nseg // blk

    def body(ma, mp, wa, ba, wp, bp, out):
        i = pl.program_id(0)
        ya = jnp.dot(ma[...], wa[...], preferred_element_type=jnp.float32)
        yp = jnp.dot(mp[...], wp[...], preferred_element_type=jnp.float32)
        cand = jnp.maximum(
            jnp.max(ya, axis=0, keepdims=True) + ba[...],
            jnp.max(yp, axis=0, keepdims=True) + bp[...],
        )
        cand = jnp.maximum(cand, 0.0)

        @pl.when(i == 0)
        def _():
            out[...] = cand

        @pl.when(i > 0)
        def _():
            out[...] = jnp.maximum(out[...], cand)

    return pl.pallas_call(
        body,
        grid=(ngrid,),
        in_specs=[
            pl.BlockSpec((blk, d), lambda i: (i, 0)),
            pl.BlockSpec((blk, d), lambda i: (i, 0)),
            pl.BlockSpec((d, out_dim), lambda i: (0, 0)),
            pl.BlockSpec((1, out_dim), lambda i: (0, 0)),
            pl.BlockSpec((d, out_dim), lambda i: (0, 0)),
            pl.BlockSpec((1, out_dim), lambda i: (0, 0)),
        ],
        out_specs=pl.BlockSpec((1, out_dim), lambda i: (0, 0)),
        out_shape=jax.ShapeDtypeStruct((1, out_dim), jnp.float32),
    )(m_a, m_p, W_a, b_a.reshape(1, -1), W_p, b_p.reshape(1, -1))


def kernel(x_author, x_paper, ptr_author, ptr_paper,
           W_author, b_author, W_paper, b_paper):
    n, d = x_author.shape
    s = ptr_author.shape[0] - 1
    out_dim = W_author.shape[1]
    n_workers = 32
    seg_per_w = (-(-s // n_workers) + 7) // 8 * 8
    nseg_pad = n_workers * seg_per_w
    tw = seg_per_w + 24
    chunk = 256

    pad = jnp.full((nseg_pad + 24 - (s + 1),), n, jnp.int32)
    ptr_a = jnp.concatenate([ptr_author.astype(jnp.int32), pad])
    ptr_p = jnp.concatenate([ptr_paper.astype(jnp.int32), pad])

    m_a, m_p = _sc_segmax(x_author, x_paper, ptr_a, ptr_p,
                          n=n, d=d, seg_per_w=seg_per_w, chunk=chunk,
                          n_workers=n_workers, tw=tw)

    blk = 1000 if s % 1000 == 0 else 8
    out = _tc_proj_reduce(m_a, m_p, W_author, b_author, W_paper, b_paper,
                          nseg=s, d=d, out_dim=out_dim, blk=blk)
    return out.reshape(out_dim)


# speedup vs baseline: 146.5330x; 1.0646x over previous
import jax
import jax.numpy as jnp
from jax import lax
from jax.experimental import pallas as pl
from jax.experimental.pallas import tpu as pltpu
from jax.experimental.pallas import tpu_sc as plsc

_LANES = 16


def _sc_segmax(x_a, x_p, ptr_a, ptr_p, *, n, d, seg_per_w, chunk,
               n_workers, tw, nseg_real):
    nseg_pad = n_workers * seg_per_w
    kg = d // _LANES
    mesh = plsc.VectorSubcoreMesh(core_axis_name="c", subcore_axis_name="s")
    num_cores = 2
    bsteps = max(1, (tw - 1).bit_length())

    def pscal(ref, i):
        return ref[pl.ds(i, _LANES)][0]

    def body(xa_hbm, xp_hbm, pa_hbm, pp_hbm, ma_hbm, mp_hbm,
             ptr_v, buf0_v, buf1_v, m_v, his_v, ids_v, sem0, sem1, semo):
        wid = lax.axis_index("s") * num_cores + lax.axis_index("c")
        s0 = pl.multiple_of(wid * seg_per_w, 8)
        neg = jnp.full((_LANES,), -jnp.inf, jnp.float32)
        zero = jnp.zeros((_LANES,), jnp.float32)
        zlim = jnp.clip(nseg_real - wid * seg_per_w, 0, seg_per_w)

        def reduce_rows(buf_v, base, g0, g1, acc):
            def rbody(g, a):
                idx = g - base
                return tuple(
                    jnp.maximum(a[q], buf_v[idx, pl.ds(_LANES * q, _LANES)])
                    for q in range(kg))
            return lax.fori_loop(g0, g1, rbody, acc)

        prev_out = None
        for x_hbm, p_hbm, m_hbm in ((xa_hbm, pa_hbm, ma_hbm),
                                    (xp_hbm, pp_hbm, mp_hbm)):
            pltpu.sync_copy(p_hbm.at[pl.ds(s0, seg_per_w + 24)], ptr_v)
            rlo = pscal(ptr_v, 0)
            rhi = pscal(ptr_v, seg_per_w)
            nch = jnp.maximum((rhi - rlo + chunk - 1) // chunk, 1)

            def chunk_base(c):
                clo = rlo + c * chunk
                return pl.multiple_of(
                    jnp.minimum(clo - clo % 8, n - (chunk + 8)), 8)

            def dma_start(c, buf, sem):
                pltpu.make_async_copy(
                    x_hbm.at[pl.ds(chunk_base(c), chunk + 8), :], buf,
                    sem).start()

            def dma_wait(buf, sem):
                pltpu.make_async_copy(
                    x_hbm.at[pl.ds(0, chunk + 8), :], buf, sem).wait()

            dma_start(0, buf0_v, sem0)
            dma_start(1, buf1_v, sem1)

            def tinit(j, _):
                his_v[j] = n + 1
                return 0
            lax.fori_loop(0, tw, tinit, 0)

            def tbody(s_, carry):
                k_, lo_ = carry
                hi_ = pscal(ptr_v, s_ + 1)

                def t_yes(k__):
                    his_v[k__] = hi_
                    ids_v[k__] = s_
                    return k__ + 1

                k_ = lax.cond(hi_ > lo_, t_yes, lambda k__: k__, k_)
                return (k_, hi_)

            lax.fori_loop(0, seg_per_w, tbody, (jnp.int32(0), rlo))

            if prev_out is not None:
                prev_out.wait()

            def process(c, buf_v, carry):
                klo, r, prev = carry[0], carry[1], carry[2]
                acc = tuple(carry[3:])
                base = chunk_base(c)
                clim = jnp.minimum(rlo + c * chunk + chunk, rhi)

                blo = jnp.int32(0)
                bhi = jnp.int32(tw)
                for _ in range(bsteps):
                    mid = (blo + bhi) // 2
                    gt = his_v[mid] > clim
                    live = blo < bhi
                    blo = jnp.where(jnp.logical_and(live, ~gt), mid + 1, blo)
                    bhi = jnp.where(jnp.logical_and(live, gt), mid, bhi)
                khi = blo

                def seg_body(k, st):
                    start, prev_ = st[0], st[1]
                    a = tuple(st[2:])
                    hi_k = his_v[k]
                    a = reduce_rows(buf_v, base, start, hi_k, a)
                    row = ids_v[k]

                    def zfill(j, _):
                        for q in range(kg):
                            m_v[j, pl.ds(_LANES * q, _LANES)] = zero
                        return 0
                    lax.fori_loop(prev_ + 1, row, zfill, 0)

                    for q in range(kg):
                        m_v[row, pl.ds(_LANES * q, _LANES)] = a[q]
                    return (hi_k, row) + tuple(neg for _ in range(kg))

                st = lax.fori_loop(klo, khi, seg_body, (r, prev) + acc)
                start, prev = st[0], st[1]
                acc = tuple(st[2:])
                acc = reduce_rows(buf_v, base, start, clim, acc)
                return (khi, jnp.maximum(clim, r), prev) + acc

            nh = (nch + 1) // 2

            def pair_body(h, carry):
                dma_wait(buf0_v, sem0)
                carry = process(2 * h, buf0_v, carry)
                dma_start(2 * h + 2, buf0_v, sem0)
                dma_wait(buf1_v, sem1)
                carry = process(2 * h + 1, buf1_v, carry)
                dma_start(2 * h + 3, buf1_v, sem1)
                return carry

            init = (jnp.int32(0), rlo, jnp.int32(-1)) + tuple(
                neg for _ in range(kg))
            fin = lax.fori_loop(0, nh, pair_body, init)
            dma_wait(buf0_v, sem0)
            dma_wait(buf1_v, sem1)
            prev = fin[2]

            def zfill_tail(j, _):
                for q in range(kg):
                    m_v[j, pl.ds(_LANES * q, _LANES)] = zero
                return 0
            lax.fori_loop(prev + 1, zlim, zfill_tail, 0)

            prev_out = pltpu.make_async_copy(
                m_v, m_hbm.at[pl.ds(s0, seg_per_w), :], semo)
            prev_out.start()
        prev_out.wait()

    f = pl.kernel(
        body,
        out_type=(
            jax.ShapeDtypeStruct((nseg_pad, d), jnp.float32),
            jax.ShapeDtypeStruct((nseg_pad, d), jnp.float32),
        ),
        mesh=mesh,
        scratch_types=[
            pltpu.VMEM((seg_per_w + 24,), jnp.int32),
            pltpu.VMEM((chunk + 8, d), jnp.float32),
            pltpu.VMEM((chunk + 8, d), jnp.float32),
            pltpu.VMEM((seg_per_w, d), jnp.float32),
            pltpu.SMEM((tw,), jnp.int32),
            pltpu.SMEM((tw,), jnp.int32),
            pltpu.SemaphoreType.DMA,
            pltpu.SemaphoreType.DMA,
            pltpu.SemaphoreType.DMA,
        ],
    )
    return f(x_a, x_p, ptr_a, ptr_p)


def _tc_proj_reduce(m_a, m_p, W_a, b_a, W_p, b_p, *, nseg, d, out_dim, blk):
    ngrid = nseg // blk

    def body(ma, mp, wa, ba, wp, bp, out):
        i = pl.program_id(0)
        ya = jnp.dot(ma[...], wa[...], preferred_element_type=jnp.float32)
        yp = jnp.dot(mp[...], wp[...], preferred_element_type=jnp.float32)
        cand = jnp.maximum(
            jnp.max(ya, axis=0, keepdims=True) + ba[...],
            jnp.max(yp, axis=0, keepdims=True) + bp[...],
        )
        cand = jnp.maximum(cand, 0.0)

        @pl.when(i == 0)
        def _():
            out[...] = cand

        @pl.when(i > 0)
        def _():
            out[...] = jnp.maximum(out[...], cand)

    return pl.pallas_call(
        body,
        grid=(ngrid,),
        in_specs=[
            pl.BlockSpec((blk, d), lambda i: (i, 0)),
            pl.BlockSpec((blk, d), lambda i: (i, 0)),
            pl.BlockSpec((d, out_dim), lambda i: (0, 0)),
            pl.BlockSpec((1, out_dim), lambda i: (0, 0)),
            pl.BlockSpec((d, out_dim), lambda i: (0, 0)),
            pl.BlockSpec((1, out_dim), lambda i: (0, 0)),
        ],
        out_specs=pl.BlockSpec((1, out_dim), lambda i: (0, 0)),
        out_shape=jax.ShapeDtypeStruct((1, out_dim), jnp.float32),
    )(m_a, m_p, W_a, b_a.reshape(1, -1), W_p, b_p.reshape(1, -1))


def kernel(x_author, x_paper, ptr_author, ptr_paper,
           W_author, b_author, W_paper, b_paper):
    n, d = x_author.shape
    s = ptr_author.shape[0] - 1
    out_dim = W_author.shape[1]
    n_workers = 32
    seg_per_w = (-(-s // n_workers) + 7) // 8 * 8
    nseg_pad = n_workers * seg_per_w
    tw = seg_per_w + 32
    chunk = 256

    pad = jnp.full((nseg_pad + 24 - (s + 1),), n, jnp.int32)
    ptr_a = jnp.concatenate([ptr_author.astype(jnp.int32), pad])
    ptr_p = jnp.concatenate([ptr_paper.astype(jnp.int32), pad])

    m_a, m_p = _sc_segmax(x_author, x_paper, ptr_a, ptr_p,
                          n=n, d=d, seg_per_w=seg_per_w, chunk=chunk,
                          n_workers=n_workers, tw=tw, nseg_real=s)

    blk = 1000 if s % 1000 == 0 else 8
    out = _tc_proj_reduce(m_a, m_p, W_author, b_author, W_paper, b_paper,
                          nseg=s, d=d, out_dim=out_dim, blk=blk)
    return out.reshape(out_dim)


# speedup vs baseline: 156.3394x; 1.0669x over previous
import jax
import jax.numpy as jnp
from jax import lax
from jax.experimental import pallas as pl
from jax.experimental.pallas import tpu as pltpu
from jax.experimental.pallas import tpu_sc as plsc

_LANES = 16


def _sc_segmax(x_a, x_p, ptr_a, ptr_p, *, n, d, seg_per_w, chunk,
               n_workers, tw, nseg_real):
    nseg_pad = n_workers * seg_per_w
    kg = d // _LANES
    mesh = plsc.VectorSubcoreMesh(core_axis_name="c", subcore_axis_name="s")
    num_cores = 2
    bsteps = max(1, (tw - 1).bit_length())

    def pscal(ref, i):
        return ref[pl.ds(i, _LANES)][0]

    def body(xa_hbm, xp_hbm, pa_hbm, pp_hbm, ma_hbm, mp_hbm,
             ptr_v, buf0_v, buf1_v, m_v, his_v, ids_v, sem0, sem1, semo):
        wid = lax.axis_index("s") * num_cores + lax.axis_index("c")
        s0 = pl.multiple_of(wid * seg_per_w, 8)
        neg = jnp.full((_LANES,), -jnp.inf, jnp.float32)
        zero = jnp.zeros((_LANES,), jnp.float32)
        zlim = jnp.clip(nseg_real - wid * seg_per_w, 0, seg_per_w)

        def reduce_rows(buf_v, base, g0, g1, acc):
            def rbody(g, a):
                idx = g - base
                return tuple(
                    jnp.maximum(a[q], buf_v[idx, pl.ds(_LANES * q, _LANES)])
                    for q in range(kg))
            return lax.fori_loop(g0, g1, rbody, acc)

        prev_out = None
        for x_hbm, p_hbm, m_hbm in ((xa_hbm, pa_hbm, ma_hbm),
                                    (xp_hbm, pp_hbm, mp_hbm)):
            pltpu.sync_copy(p_hbm.at[pl.ds(s0, seg_per_w + 24)], ptr_v)
            rlo = pscal(ptr_v, 0)
            rhi = pscal(ptr_v, seg_per_w)
            nch = jnp.maximum((rhi - rlo + chunk - 1) // chunk, 1)

            def chunk_base(c):
                clo = rlo + c * chunk
                return pl.multiple_of(
                    jnp.minimum(clo - clo % 8, n - (chunk + 8)), 8)

            def dma_start(c, buf, sem):
                @pl.when(c < nch)
                def _():
                    pltpu.make_async_copy(
                        x_hbm.at[pl.ds(chunk_base(c), chunk + 8), :], buf,
                        sem).start()

            def dma_wait(c, buf, sem):
                @pl.when(c < nch)
                def _():
                    pltpu.make_async_copy(
                        x_hbm.at[pl.ds(0, chunk + 8), :], buf, sem).wait()

            dma_start(0, buf0_v, sem0)
            dma_start(1, buf1_v, sem1)

            def tinit(j, _):
                his_v[j] = n + 1
                return 0
            lax.fori_loop(0, tw, tinit, 0)

            def tbody(s_, carry):
                k_, lo_ = carry
                hi_ = pscal(ptr_v, s_ + 1)

                def t_yes(k__):
                    his_v[k__] = hi_
                    ids_v[k__] = s_
                    return k__ + 1

                k_ = lax.cond(hi_ > lo_, t_yes, lambda k__: k__, k_)
                return (k_, hi_)

            lax.fori_loop(0, seg_per_w, tbody, (jnp.int32(0), rlo))

            if prev_out is not None:
                prev_out.wait()

            def process(c, buf_v, carry):
                klo, r, prev = carry[0], carry[1], carry[2]
                acc = tuple(carry[3:])
                base = chunk_base(c)
                clim = jnp.minimum(rlo + c * chunk + chunk, rhi)

                blo = jnp.int32(0)
                bhi = jnp.int32(tw)
                for _ in range(bsteps):
                    mid = (blo + bhi) // 2
                    gt = his_v[mid] > clim
                    live = blo < bhi
                    blo = jnp.where(jnp.logical_and(live, ~gt), mid + 1, blo)
                    bhi = jnp.where(jnp.logical_and(live, gt), mid, bhi)
                khi = blo

                def seg_body(k, st):
                    start, prev_ = st[0], st[1]
                    a = tuple(st[2:])
                    hi_k = his_v[k]
                    a = reduce_rows(buf_v, base, start, hi_k, a)
                    row = ids_v[k]

                    def zfill(j, _):
                        for q in range(kg):
                            m_v[j, pl.ds(_LANES * q, _LANES)] = zero
                        return 0
                    lax.fori_loop(prev_ + 1, row, zfill, 0)

                    for q in range(kg):
                        m_v[row, pl.ds(_LANES * q, _LANES)] = a[q]
                    return (hi_k, row) + tuple(neg for _ in range(kg))

                st = lax.fori_loop(klo, khi, seg_body, (r, prev) + acc)
                start, prev = st[0], st[1]
                acc = tuple(st[2:])
                acc = reduce_rows(buf_v, base, start, clim, acc)
                return (khi, jnp.maximum(clim, r), prev) + acc

            nh = (nch + 1) // 2

            def pair_body(h, carry):
                dma_wait(2 * h, buf0_v, sem0)
                carry = process(2 * h, buf0_v, carry)
                dma_start(2 * h + 2, buf0_v, sem0)
                dma_wait(2 * h + 1, buf1_v, sem1)
                carry = process(2 * h + 1, buf1_v, carry)
                dma_start(2 * h + 3, buf1_v, sem1)
                return carry

            init = (jnp.int32(0), rlo, jnp.int32(-1)) + tuple(
                neg for _ in range(kg))
            fin = lax.fori_loop(0, nh, pair_body, init)
            prev = fin[2]

            def zfill_tail(j, _):
                for q in range(kg):
                    m_v[j, pl.ds(_LANES * q, _LANES)] = zero
                return 0
            lax.fori_loop(prev + 1, zlim, zfill_tail, 0)

            prev_out = pltpu.make_async_copy(
                m_v, m_hbm.at[pl.ds(s0, seg_per_w), :], semo)
            prev_out.start()
        prev_out.wait()

    f = pl.kernel(
        body,
        out_type=(
            jax.ShapeDtypeStruct((nseg_pad, d), jnp.float32),
            jax.ShapeDtypeStruct((nseg_pad, d), jnp.float32),
        ),
        mesh=mesh,
        scratch_types=[
            pltpu.VMEM((seg_per_w + 24,), jnp.int32),
            pltpu.VMEM((chunk + 8, d), jnp.float32),
            pltpu.VMEM((chunk + 8, d), jnp.float32),
            pltpu.VMEM((seg_per_w, d), jnp.float32),
            pltpu.SMEM((tw,), jnp.int32),
            pltpu.SMEM((tw,), jnp.int32),
            pltpu.SemaphoreType.DMA,
            pltpu.SemaphoreType.DMA,
            pltpu.SemaphoreType.DMA,
        ],
    )
    return f(x_a, x_p, ptr_a, ptr_p)


def _tc_proj_reduce(m_a, m_p, W_a, b_a, W_p, b_p, *, nseg, d, out_dim, blk):
    ngrid = nseg // blk

    def body(ma, mp, wa, ba, wp, bp, out):
        i = pl.program_id(0)
        ya = jnp.dot(ma[...], wa[...], preferred_element_type=jnp.float32)
        yp = jnp.dot(mp[...], wp[...], preferred_element_type=jnp.float32)
        cand = jnp.maximum(
            jnp.max(ya, axis=0, keepdims=True) + ba[...],
            jnp.max(yp, axis=0, keepdims=True) + bp[...],
        )
        cand = jnp.maximum(cand, 0.0)

        @pl.when(i == 0)
        def _():
            out[...] = cand

        @pl.when(i > 0)
        def _():
            out[...] = jnp.maximum(out[...], cand)

    return pl.pallas_call(
        body,
        grid=(ngrid,),
        in_specs=[
            pl.BlockSpec((blk, d), lambda i: (i, 0)),
            pl.BlockSpec((blk, d), lambda i: (i, 0)),
            pl.BlockSpec((d, out_dim), lambda i: (0, 0)),
            pl.BlockSpec((1, out_dim), lambda i: (0, 0)),
            pl.BlockSpec((d, out_dim), lambda i: (0, 0)),
            pl.BlockSpec((1, out_dim), lambda i: (0, 0)),
        ],
        out_specs=pl.BlockSpec((1, out_dim), lambda i: (0, 0)),
        out_shape=jax.ShapeDtypeStruct((1, out_dim), jnp.float32),
    )(m_a, m_p, W_a, b_a.reshape(1, -1), W_p, b_p.reshape(1, -1))


def kernel(x_author, x_paper, ptr_author, ptr_paper,
           W_author, b_author, W_paper, b_paper):
    n, d = x_author.shape
    s = ptr_author.shape[0] - 1
    out_dim = W_author.shape[1]
    n_workers = 32
    seg_per_w = (-(-s // n_workers) + 7) // 8 * 8
    nseg_pad = n_workers * seg_per_w
    tw = seg_per_w + 32
    chunk = 256

    pad = jnp.full((nseg_pad + 24 - (s + 1),), n, jnp.int32)
    ptr_a = jnp.concatenate([ptr_author.astype(jnp.int32), pad])
    ptr_p = jnp.concatenate([ptr_paper.astype(jnp.int32), pad])

    m_a, m_p = _sc_segmax(x_author, x_paper, ptr_a, ptr_p,
                          n=n, d=d, seg_per_w=seg_per_w, chunk=chunk,
                          n_workers=n_workers, tw=tw, nseg_real=s)

    blk = 1000 if s % 1000 == 0 else 8
    out = _tc_proj_reduce(m_a, m_p, W_author, b_author, W_paper, b_paper,
                          nseg=s, d=d, out_dim=out_dim, blk=blk)
    return out.reshape(out_dim)
